# baseline (speedup 1.0000x reference)
import math

import jax
import jax.numpy as jnp
from jax.experimental import pallas as pl
from jax.experimental.pallas import tpu as pltpu

N_ROWS = 1024
N_COLS = 100000
CR = 8
NCHUNK = N_ROWS // CR
NBUF = 8
S0_SCALE = math.sqrt(2.0) * math.log(N_COLS - 1)
BETA = 1.0


def _stream(hbm_ref, buf, sem, body):
    def issue(c, slot):
        pltpu.make_async_copy(
            hbm_ref.at[pl.ds(c * CR, CR), :], buf.at[slot], sem.at[slot]
        ).start()

    for s in range(NBUF):
        issue(s, s)

    def outer(g, _):
        for s in range(NBUF):
            c = g * NBUF + s
            pltpu.make_async_copy(
                hbm_ref.at[pl.ds(c * CR, CR), :], buf.at[s], sem.at[s]
            ).wait()
            body(c, buf.at[s])

            @pl.when(c + NBUF < NCHUNK)
            def _():
                issue(c + NBUF, s)

        return 0

    jax.lax.fori_loop(0, NCHUNK // NBUF, outer, 0)


def _pass1_kernel(lab_ref, hbm_ref, s0_ref, t_ref, buf, sem):
    def body(c, chunk):
        x = chunk[...]
        lab = lab_ref[pl.ds(c * CR, CR), :]
        cols = jax.lax.broadcasted_iota(jnp.int32, (CR, N_COLS), 1)
        e0 = jnp.exp(S0_SCALE * x)
        s0_ref[pl.ds(c * CR, CR), :] = jnp.sum(e0, axis=1, keepdims=True)
        tp = jnp.max(jnp.where(cols == lab, x, -2.0), axis=1, keepdims=True)
        t_ref[pl.ds(c * CR, CR), :] = tp

    _stream(hbm_ref, buf, sem, body)


def _pass2_kernel(s_ref, hbm_ref, s1_ref, ss_ref, buf, sem):
    s = s_ref[0, 0]

    def body(c, chunk):
        x = chunk[...]
        s1_ref[pl.ds(c * CR, CR), :] = jnp.sum(jnp.exp(x), axis=1, keepdims=True)
        ss_ref[pl.ds(c * CR, CR), :] = jnp.sum(
            jnp.exp(s * x), axis=1, keepdims=True
        )

    _stream(hbm_ref, buf, sem, body)


def _acos(x):
    ax = jnp.abs(x)
    p = jnp.float32(-0.0012624911)
    for c in (0.0066700901, -0.0170881256, 0.0308918810, -0.0501743046,
              0.0889789874, -0.2145988016, 1.5707963050):
        p = p * ax + jnp.float32(c)
    r = jnp.sqrt(jnp.maximum(0.0, 1.0 - ax)) * p
    return jnp.where(x >= 0.0, r, jnp.float32(math.pi) - r)


def _kth_smallest(c, k, n_iter=48):
    def body(_, carry):
        lo, hi = carry
        mid = 0.5 * (lo + hi)
        cnt = jnp.sum((c <= mid).astype(jnp.float32))
        take_hi = cnt >= (k + 1)
        return (jnp.where(take_hi, lo, mid), jnp.where(take_hi, mid, hi))

    lo, hi = jax.lax.fori_loop(
        0, n_iter, body, (jnp.float32(-1.1), jnp.float32(1.1))
    )
    return hi


def _mid_kernel(s0_ref, t_ref, s_out, mt_out):
    t = t_ref[...]
    sum0 = jnp.sum(s0_ref[...]) - jnp.sum(jnp.exp(S0_SCALE * t))
    b_avg = sum0 / N_ROWS
    c = jnp.clip(t, -1.0 + 1e-07, 1.0 - 1e-07)
    ca = _kth_smallest(c, N_ROWS // 2 - 1)
    cb = _kth_smallest(c, N_ROWS // 2)
    theta_med = 0.5 * (_acos(ca) + _acos(cb))
    cos_sum = ca * cb - jnp.sqrt(
        jnp.maximum(0.0, (1.0 - ca * ca)) * jnp.maximum(0.0, (1.0 - cb * cb))
    )
    cos_med = jnp.sqrt(jnp.maximum(0.0, 0.5 * (1.0 + cos_sum)))
    denom = jnp.where(
        theta_med < jnp.float32(math.pi / 4.0),
        cos_med,
        jnp.float32(math.cos(math.pi / 4.0)),
    )
    s = jnp.log(b_avg) / denom
    s_out[...] = jnp.reshape(s, (1, 1))
    mt_out[...] = jnp.reshape(jnp.mean(t), (1, 1))


def _final_kernel(s1_ref, ss_ref, t_ref, s_ref, out_ref):
    t = t_ref[...]
    s = s_ref[...]
    loss1 = jnp.mean(jnp.log(ss_ref[...])) - s * jnp.mean(t)
    loss2 = jnp.mean(jnp.log(s1_ref[...])) - jnp.mean(t)
    out_ref[...] = (BETA * loss1 + loss2) / (1.0 + BETA)


def kernel(logits, labels):
    labels2 = labels.astype(jnp.int32).reshape(N_ROWS, 1)

    colvec = jax.ShapeDtypeStruct((N_ROWS, 1), jnp.float32)
    scalar = jax.ShapeDtypeStruct((1, 1), jnp.float32)
    scratch = [
        pltpu.VMEM((NBUF, CR, N_COLS), jnp.float32),
        pltpu.SemaphoreType.DMA((NBUF,)),
    ]

    s0_rows, t_rows = pl.pallas_call(
        _pass1_kernel,
        in_specs=[
            pl.BlockSpec(memory_space=pltpu.VMEM),
            pl.BlockSpec(memory_space=pl.ANY),
        ],
        out_specs=[pl.BlockSpec(memory_space=pltpu.VMEM)] * 2,
        out_shape=[colvec, colvec],
        scratch_shapes=scratch,
    )(labels2, logits)

    s_sc, mt_sc = pl.pallas_call(
        _mid_kernel,
        out_shape=[scalar, scalar],
    )(s0_rows, t_rows)
    del mt_sc

    s1_rows, ss_rows = pl.pallas_call(
        _pass2_kernel,
        in_specs=[
            pl.BlockSpec(memory_space=pltpu.SMEM),
            pl.BlockSpec(memory_space=pl.ANY),
        ],
        out_specs=[pl.BlockSpec(memory_space=pltpu.VMEM)] * 2,
        out_shape=[colvec, colvec],
        scratch_shapes=scratch,
    )(s_sc, logits)

    loss = pl.pallas_call(
        _final_kernel,
        out_shape=scalar,
    )(s1_rows, ss_rows, t_rows, s_sc)

    return loss[0, 0]


# speedup vs baseline: 2.4892x; 2.4892x over previous
import math

import jax
import jax.numpy as jnp
from jax.experimental import pallas as pl
from jax.experimental.pallas import tpu as pltpu

N_ROWS = 1024
N_COLS = 100000
KC = 1000
NCHUNK = N_COLS // KC
NBUF = 8
S0_SCALE = math.sqrt(2.0) * math.log(N_COLS - 1)
BETA = 1.0


def _stream(hbm_ref, buf, sem, body):
    def issue(c, slot):
        pltpu.make_async_copy(
            hbm_ref.at[pl.ds(c * KC, KC), :], buf.at[slot], sem.at[slot]
        ).start()

    for s in range(NBUF):
        issue(s, s)

    def outer(g, _):
        for s in range(NBUF):
            c = g * NBUF + s
            pltpu.make_async_copy(
                hbm_ref.at[pl.ds(c * KC, KC), :], buf.at[s], sem.at[s]
            ).wait()
            body(c, buf.at[s])

            @pl.when(c + NBUF < NCHUNK)
            def _():
                issue(c + NBUF, s)

        return 0

    n_groups = (NCHUNK + NBUF - 1) // NBUF
    if NCHUNK % NBUF == 0:
        jax.lax.fori_loop(0, n_groups, outer, 0)
    else:
        jax.lax.fori_loop(0, n_groups - 1, outer, 0)
        g = n_groups - 1
        for s in range(NCHUNK - (n_groups - 1) * NBUF):
            c = g * NBUF + s
            pltpu.make_async_copy(
                hbm_ref.at[pl.ds(c * KC, KC), :], buf.at[s], sem.at[s]
            ).wait()
            body(c, buf.at[s])


def _pass1_kernel(lab_ref, hbm_ref, s0_ref, t_ref, buf, sem):
    lab = lab_ref[...]
    s0_ref[...] = jnp.zeros_like(s0_ref)
    t_ref[...] = jnp.full_like(t_ref, -2.0)

    def body(c, chunk):
        x = chunk[...]
        cls = jax.lax.broadcasted_iota(jnp.int32, (KC, N_ROWS), 0) + c * KC
        e0 = jnp.exp(S0_SCALE * x)
        s0_ref[...] += jnp.sum(e0, axis=0, keepdims=True)
        tp = jnp.max(jnp.where(cls == lab, x, -2.0), axis=0, keepdims=True)
        t_ref[...] = jnp.maximum(t_ref[...], tp)

    _stream(hbm_ref, buf, sem, body)


def _pass2_kernel(s_ref, hbm_ref, s1_ref, ss_ref, buf, sem):
    s = s_ref[0, 0]
    s1_ref[...] = jnp.zeros_like(s1_ref)
    ss_ref[...] = jnp.zeros_like(ss_ref)

    def body(c, chunk):
        x = chunk[...]
        s1_ref[...] += jnp.sum(jnp.exp(x), axis=0, keepdims=True)
        ss_ref[...] += jnp.sum(jnp.exp(s * x), axis=0, keepdims=True)

    _stream(hbm_ref, buf, sem, body)


def _acos(x):
    ax = jnp.abs(x)
    p = jnp.float32(-0.0012624911)
    for c in (0.0066700901, -0.0170881256, 0.0308918810, -0.0501743046,
              0.0889789874, -0.2145988016, 1.5707963050):
        p = p * ax + jnp.float32(c)
    r = jnp.sqrt(jnp.maximum(0.0, 1.0 - ax)) * p
    return jnp.where(x >= 0.0, r, jnp.float32(math.pi) - r)


def _two_kth_smallest(c, ka, kb, n_iter=48):
    def body(_, carry):
        lo_a, hi_a, lo_b, hi_b = carry
        mid_a = 0.5 * (lo_a + hi_a)
        mid_b = 0.5 * (lo_b + hi_b)
        cnt_a = jnp.sum((c <= mid_a).astype(jnp.float32))
        cnt_b = jnp.sum((c <= mid_b).astype(jnp.float32))
        ta = cnt_a >= (ka + 1)
        tb = cnt_b >= (kb + 1)
        return (
            jnp.where(ta, lo_a, mid_a), jnp.where(ta, mid_a, hi_a),
            jnp.where(tb, lo_b, mid_b), jnp.where(tb, mid_b, hi_b),
        )

    init = (jnp.float32(-1.1), jnp.float32(1.1),
            jnp.float32(-1.1), jnp.float32(1.1))
    _, hi_a, _, hi_b = jax.lax.fori_loop(0, n_iter, body, init)
    return hi_a, hi_b


def _mid_kernel(s0_ref, t_ref, s_out, mt_out):
    t = t_ref[...]
    sum0 = jnp.sum(s0_ref[...]) - jnp.sum(jnp.exp(S0_SCALE * t))
    b_avg = sum0 / N_ROWS
    c = jnp.clip(t, -1.0 + 1e-07, 1.0 - 1e-07)
    ca, cb = _two_kth_smallest(c, N_ROWS // 2 - 1, N_ROWS // 2)
    theta_med = 0.5 * (_acos(ca) + _acos(cb))
    cos_sum = ca * cb - jnp.sqrt(
        jnp.maximum(0.0, (1.0 - ca * ca)) * jnp.maximum(0.0, (1.0 - cb * cb))
    )
    cos_med = jnp.sqrt(jnp.maximum(0.0, 0.5 * (1.0 + cos_sum)))
    denom = jnp.where(
        theta_med < jnp.float32(math.pi / 4.0),
        cos_med,
        jnp.float32(math.cos(math.pi / 4.0)),
    )
    s = jnp.log(b_avg) / denom
    s_out[...] = jnp.reshape(s, (1, 1))
    mt_out[...] = jnp.reshape(jnp.mean(t), (1, 1))


def _final_kernel(s1_ref, ss_ref, t_ref, s_ref, out_ref):
    t = t_ref[...]
    s = s_ref[...]
    loss1 = jnp.mean(jnp.log(ss_ref[...])) - s * jnp.mean(t)
    loss2 = jnp.mean(jnp.log(s1_ref[...])) - jnp.mean(t)
    out_ref[...] = (BETA * loss1 + loss2) / (1.0 + BETA)


def kernel(logits, labels):
    xt = logits.T
    labels2 = labels.astype(jnp.int32).reshape(1, N_ROWS)

    rowvec = jax.ShapeDtypeStruct((1, N_ROWS), jnp.float32)
    scalar = jax.ShapeDtypeStruct((1, 1), jnp.float32)
    scratch = [
        pltpu.VMEM((NBUF, KC, N_ROWS), jnp.float32),
        pltpu.SemaphoreType.DMA((NBUF,)),
    ]

    s0_rows, t_rows = pl.pallas_call(
        _pass1_kernel,
        in_specs=[
            pl.BlockSpec(memory_space=pltpu.VMEM),
            pl.BlockSpec(memory_space=pl.ANY),
        ],
        out_specs=[pl.BlockSpec(memory_space=pltpu.VMEM)] * 2,
        out_shape=[rowvec, rowvec],
        scratch_shapes=scratch,
    )(labels2, xt)

    s_sc, mt_sc = pl.pallas_call(
        _mid_kernel,
        out_shape=[scalar, scalar],
    )(s0_rows, t_rows)
    del mt_sc

    s1_rows, ss_rows = pl.pallas_call(
        _pass2_kernel,
        in_specs=[
            pl.BlockSpec(memory_space=pltpu.SMEM),
            pl.BlockSpec(memory_space=pl.ANY),
        ],
        out_specs=[pl.BlockSpec(memory_space=pltpu.VMEM)] * 2,
        out_shape=[rowvec, rowvec],
        scratch_shapes=scratch,
    )(s_sc, xt)

    loss = pl.pallas_call(
        _final_kernel,
        out_shape=scalar,
    )(s1_rows, ss_rows, t_rows, s_sc)

    return loss[0, 0]
